# baseline (speedup 1.0000x reference)
import functools

import jax
import jax.numpy as jnp
from jax import lax
from jax.experimental import pallas as pl
from jax.experimental.pallas import tpu as pltpu
from jax.experimental.pallas import tpu_sc as plsc

N_COARSE = 2048
N_FINE = 8192
C_IN = 256
C_SKIP = 128
K = 3
EPS = 1e-5

QB = 2048
NBLK = N_FINE // QB

NC = 2
NS = 16
NW = NC * NS
QPW = N_FINE // NW
SUB = 128


QBK = 2048
NBLKK = N_FINE // QBK


def _knn_body(ps_ref, bs_ref, pt_ref, bc_ref, idx_ref, w_ref):
    ps = ps_ref[...]
    pt = pt_ref[...]
    q2 = jnp.sum(ps * ps, axis=1, keepdims=True)
    c2 = jnp.sum(pt * pt, axis=0, keepdims=True)
    cross = jnp.dot(ps, pt, preferred_element_type=jnp.float32)
    d = q2 + c2 - 2.0 * cross
    mask = bs_ref[...].astype(jnp.float32) != bc_ref[...].astype(jnp.float32)
    d = jnp.where(mask, jnp.float32(1e10), d)

    fiota = lax.broadcasted_iota(jnp.int32, (QBK, N_COARSE), 1).astype(
        jnp.float32)
    idxs = []
    ws = []
    for k in range(K):
        m = jnp.min(d, axis=1, keepdims=True)
        fi = jnp.min(jnp.where(d == m, fiota, jnp.float32(N_COARSE)),
                     axis=1, keepdims=True)
        idxs.append(fi.astype(jnp.int32))
        ws.append(1.0 / jnp.maximum(m, jnp.float32(1e-16)))
        if k < K - 1:
            d = jnp.where(fiota == fi, jnp.float32(jnp.inf), d)
    wsum = ws[0] + ws[1] + ws[2]
    idx_ref[...] = jnp.concatenate(idxs, axis=1)
    w_ref[...] = jnp.concatenate([w / wsum for w in ws], axis=1)


def _knn_topk(pos_skip, batch_skip_2d, pos_t, batch_2d):
    m = pos_skip.shape[0]
    return pl.pallas_call(
        _knn_body,
        grid=(m // QBK,),
        in_specs=[
            pl.BlockSpec((QBK, 3), lambda i: (i, 0)),
            pl.BlockSpec((QBK, 1), lambda i: (i, 0)),
            pl.BlockSpec((3, N_COARSE), lambda i: (0, 0)),
            pl.BlockSpec((1, N_COARSE), lambda i: (0, 0)),
        ],
        out_specs=[
            pl.BlockSpec((QBK, K), lambda i: (i, 0)),
            pl.BlockSpec((QBK, K), lambda i: (i, 0)),
        ],
        out_shape=[
            jax.ShapeDtypeStruct((m, K), jnp.int32),
            jax.ShapeDtypeStruct((m, K), jnp.float32),
        ],
    )(pos_skip, batch_skip_2d, pos_t, batch_2d)


def _sc_gather_body(m, x_hbm, idx_hbm, out_hbm, idx_bufs, rows, sems):
    nchunk = K * (m // NW) // SUB
    qpw = m // NW
    wid = lax.axis_index("s") * NC + lax.axis_index("c")
    sem_i, sg0, sg1, sw0, sw1 = sems
    sg = [sg0, sg1]
    sw = [sw0, sw1]

    def off(t):
        k, s = divmod(t, qpw // SUB)
        return k * m + wid * qpw + s * SUB

    idx_cps = [pltpu.async_copy(idx_hbm.at[pl.ds(off(t), SUB)], idx_bufs[t],
                                sem_i) for t in range(nchunk)]
    for cp in idx_cps:
        cp.wait()

    gathers = [None] * nchunk
    writes = [None] * nchunk
    gathers[0] = pltpu.async_copy(x_hbm.at[idx_bufs[0]], rows[0], sg[0])
    for t in range(nchunk):
        gathers[t].wait()
        if t >= 1:
            writes[t - 1].wait()
        writes[t] = pltpu.async_copy(rows[t % 2], out_hbm.at[pl.ds(off(t), SUB)],
                                     sw[t % 2])
        if t + 1 < nchunk:
            gathers[t + 1] = pltpu.async_copy(x_hbm.at[idx_bufs[t + 1]],
                                              rows[(t + 1) % 2], sg[(t + 1) % 2])
    writes[nchunk - 1].wait()


@functools.cache
def _sc_gather_kernel(m):
    nchunk = K * (m // NW) // SUB

    def body(x_hbm, idx_hbm, out_hbm, *refs):
        idx_bufs = list(refs[:nchunk])
        rows = list(refs[nchunk:nchunk + 2])
        sems = refs[nchunk + 2:]
        _sc_gather_body(m, x_hbm, idx_hbm, out_hbm, idx_bufs, rows, sems)

    return pl.kernel(
        body,
        out_type=jax.ShapeDtypeStruct((K * m, C_IN), jnp.float32),
        mesh=plsc.VectorSubcoreMesh(core_axis_name="c", subcore_axis_name="s",
                                    num_cores=NC, num_subcores=NS),
        scratch_types=(
            [pltpu.VMEM((SUB,), jnp.int32) for _ in range(nchunk)]
            + [pltpu.VMEM((SUB, C_IN), jnp.float32) for _ in range(2)]
            + [pltpu.SemaphoreType.DMA for _ in range(5)]
        ),
    )


def _sc_gather(x, idx_flat, m=N_FINE):
    return _sc_gather_kernel(m)(x, idx_flat)


def _bn_silu(h, s, ss, g, be):
    mu = s * (1.0 / N_FINE)
    var = ss * (1.0 / N_FINE) - mu * mu
    hn = (h - mu) * lax.rsqrt(var + EPS) * g + be
    return hn * jax.nn.sigmoid(hn)


def _mlp_body(fa0_ref, fa1_ref, fa2_ref, fb0_ref, fb1_ref, fb2_ref,
              w_ref, xs_ref, W1_ref, b1_ref,
              g1_ref, be1_ref, W2_ref, b2_ref, g2_ref, be2_ref,
              out_ref, h1_s, h2_s, s1_s, ss1_s, s2_s, ss2_s):
    p = pl.program_id(0)
    i = pl.program_id(1)
    row = i * QB

    @pl.when(p == 0)
    def _phase0():
        w = w_ref[...]
        ya = (w[:, 0:1] * fa0_ref[...] + w[:, 1:2] * fa1_ref[...]
              + w[:, 2:3] * fa2_ref[...])
        yb = (w[:, 0:1] * fb0_ref[...] + w[:, 1:2] * fb1_ref[...]
              + w[:, 2:3] * fb2_ref[...])
        y = jnp.where(i < NBLK // 2, ya, yb)
        h = (jnp.dot(y, W1_ref[0:C_IN, :],
                     preferred_element_type=jnp.float32)
             + jnp.dot(xs_ref[...], W1_ref[C_IN:C_IN + C_SKIP, :],
                       preferred_element_type=jnp.float32)
             + b1_ref[...])

        @pl.when(i == 0)
        def _init():
            s1_s[...] = jnp.zeros_like(s1_s)
            ss1_s[...] = jnp.zeros_like(ss1_s)

        h1_s[pl.ds(row, QB), :] = h
        s1_s[...] += jnp.sum(h, axis=0, keepdims=True)
        ss1_s[...] += jnp.sum(h * h, axis=0, keepdims=True)

    @pl.when(p == 1)
    def _phase1():
        a = _bn_silu(h1_s[pl.ds(row, QB), :], s1_s[...], ss1_s[...],
                     g1_ref[...], be1_ref[...])
        h = (jnp.dot(a, W2_ref[...], preferred_element_type=jnp.float32)
             + b2_ref[...])

        @pl.when(i == 0)
        def _init():
            s2_s[...] = jnp.zeros_like(s2_s)
            ss2_s[...] = jnp.zeros_like(ss2_s)

        h2_s[pl.ds(row, QB), :] = h
        s2_s[...] += jnp.sum(h, axis=0, keepdims=True)
        ss2_s[...] += jnp.sum(h * h, axis=0, keepdims=True)

    @pl.when(p == 2)
    def _phase2():
        out_ref[...] = _bn_silu(h2_s[pl.ds(row, QB), :], s2_s[...],
                                ss2_s[...], g2_ref[...], be2_ref[...])


def _mlp(fa, fb, w, x_skip, W1, b1, g1, be1, W2, b2, g2, be2):
    def blk(p, i):
        return (jnp.where(p == 0, i, 0), 0)

    HB = NBLK // 2

    def fa_map(k):
        return lambda p, i: (
            k * HB + jnp.where((p == 0) & (i < HB), i, 0), 0)

    def fb_map(k):
        return lambda p, i: (
            k * HB + jnp.where((p == 0) & (i >= HB), i - HB, 0), 0)

    return pl.pallas_call(
        _mlp_body,
        grid=(3, NBLK),
        in_specs=[
            pl.BlockSpec((QB, C_IN), fa_map(0)),
            pl.BlockSpec((QB, C_IN), fa_map(1)),
            pl.BlockSpec((QB, C_IN), fa_map(2)),
            pl.BlockSpec((QB, C_IN), fb_map(0)),
            pl.BlockSpec((QB, C_IN), fb_map(1)),
            pl.BlockSpec((QB, C_IN), fb_map(2)),
            pl.BlockSpec((QB, K), blk),
            pl.BlockSpec((QB, C_SKIP), blk),
            pl.BlockSpec((C_IN + C_SKIP, 256), lambda p, i: (0, 0)),
            pl.BlockSpec((1, 256), lambda p, i: (0, 0)),
            pl.BlockSpec((1, 256), lambda p, i: (0, 0)),
            pl.BlockSpec((1, 256), lambda p, i: (0, 0)),
            pl.BlockSpec((256, 256), lambda p, i: (0, 0)),
            pl.BlockSpec((1, 256), lambda p, i: (0, 0)),
            pl.BlockSpec((1, 256), lambda p, i: (0, 0)),
            pl.BlockSpec((1, 256), lambda p, i: (0, 0)),
        ],
        out_specs=pl.BlockSpec((QB, 256),
                               lambda p, i: (jnp.where(p == 2, i, 0), 0)),
        out_shape=jax.ShapeDtypeStruct((N_FINE, 256), jnp.float32),
        scratch_shapes=[
            pltpu.VMEM((N_FINE, 256), jnp.float32),
            pltpu.VMEM((N_FINE, 256), jnp.float32),
            pltpu.VMEM((1, 256), jnp.float32),
            pltpu.VMEM((1, 256), jnp.float32),
            pltpu.VMEM((1, 256), jnp.float32),
            pltpu.VMEM((1, 256), jnp.float32),
        ],
    )(fa, fa, fa, fb, fb, fb, w, x_skip, W1, b1, g1, be1, W2, b2, g2, be2)


def kernel(x, pos, batch, x_skip, pos_skip, batch_skip,
           W1, b1, g1, be1, W2, b2, g2, be2):
    bs_2d = batch_skip.astype(jnp.int32).reshape(N_FINE, 1)
    bc_2d = batch.astype(jnp.int32).reshape(1, N_COARSE)
    pos_t = pos.T

    HM = N_FINE // 2
    idx0, w0 = _knn_topk(pos_skip[:HM], bs_2d[:HM], pos_t, bc_2d)
    fa = _sc_gather(x, idx0.T.reshape(K * HM), HM)
    idx1, w1 = _knn_topk(pos_skip[HM:], bs_2d[HM:], pos_t, bc_2d)
    fb = _sc_gather(x, idx1.T.reshape(K * HM), HM)
    w = jnp.concatenate([w0, w1], axis=0)

    r = lambda v: v.reshape(1, 256)
    h = _mlp(fa, fb, w, x_skip, W1, r(b1), r(g1), r(be1),
             W2, r(b2), r(g2), r(be2))
    return (h, pos_skip, batch_skip)


# speedup vs baseline: 1.0337x; 1.0337x over previous
import functools

import jax
import jax.numpy as jnp
from jax import lax
from jax.experimental import pallas as pl
from jax.experimental.pallas import tpu as pltpu
from jax.experimental.pallas import tpu_sc as plsc

N_COARSE = 2048
N_FINE = 8192
C_IN = 256
C_SKIP = 128
K = 3
EPS = 1e-5

QB = 2048
NBLK = N_FINE // QB

NC = 2
NS = 16
NW = NC * NS
QPW = N_FINE // NW
SUB = 128


QBK = 2048
NBLKK = N_FINE // QBK


def _knn_body(ps_ref, bs_ref, pt_ref, bc_ref, idx_ref, w_ref):
    ps = ps_ref[...]
    pt = pt_ref[...]
    q2 = jnp.sum(ps * ps, axis=1, keepdims=True)
    c2 = jnp.sum(pt * pt, axis=0, keepdims=True)
    cross = jnp.dot(ps, pt, preferred_element_type=jnp.float32)
    d = q2 + c2 - 2.0 * cross
    mask = bs_ref[...].astype(jnp.float32) != bc_ref[...].astype(jnp.float32)
    d = jnp.where(mask, jnp.float32(1e10), d)

    fiota = lax.broadcasted_iota(jnp.int32, (QBK, N_COARSE), 1).astype(
        jnp.float32)
    idxs = []
    ws = []
    for k in range(K):
        m = jnp.min(d, axis=1, keepdims=True)
        fi = jnp.min(jnp.where(d == m, fiota, jnp.float32(N_COARSE)),
                     axis=1, keepdims=True)
        idxs.append(fi.astype(jnp.int32))
        ws.append(1.0 / jnp.maximum(m, jnp.float32(1e-16)))
        if k < K - 1:
            d = jnp.where(fiota == fi, jnp.float32(jnp.inf), d)
    wsum = ws[0] + ws[1] + ws[2]
    idx_ref[...] = jnp.concatenate(idxs, axis=1)
    w_ref[...] = jnp.concatenate([w / wsum for w in ws], axis=1)


def _knn_topk(pos_skip, batch_skip_2d, pos_t, batch_2d):
    return pl.pallas_call(
        _knn_body,
        grid=(NBLKK,),
        in_specs=[
            pl.BlockSpec((QBK, 3), lambda i: (i, 0)),
            pl.BlockSpec((QBK, 1), lambda i: (i, 0)),
            pl.BlockSpec((3, N_COARSE), lambda i: (0, 0)),
            pl.BlockSpec((1, N_COARSE), lambda i: (0, 0)),
        ],
        out_specs=[
            pl.BlockSpec((QBK, K), lambda i: (i, 0)),
            pl.BlockSpec((QBK, K), lambda i: (i, 0)),
        ],
        out_shape=[
            jax.ShapeDtypeStruct((N_FINE, K), jnp.int32),
            jax.ShapeDtypeStruct((N_FINE, K), jnp.float32),
        ],
    )(pos_skip, batch_skip_2d, pos_t, batch_2d)


_NCHUNK = K * QPW // SUB


def _sc_gather_body(x_hbm, idx_hbm, out_hbm,
                    i0, i1, i2, i3, i4, i5, rows0, rows1,
                    sem_i, sg0, sg1, sw0, sw1):
    wid = lax.axis_index("s") * NC + lax.axis_index("c")
    idx_bufs = [i0, i1, i2, i3, i4, i5]
    rows = [rows0, rows1]
    sg = [sg0, sg1]
    sw = [sw0, sw1]

    def off(t):
        k, s = divmod(t, QPW // SUB)
        return k * N_FINE + wid * QPW + s * SUB

    idx_cps = [pltpu.async_copy(idx_hbm.at[pl.ds(off(t), SUB)], idx_bufs[t],
                                sem_i) for t in range(_NCHUNK)]
    for cp in idx_cps:
        cp.wait()

    gathers = [None] * _NCHUNK
    writes = [None] * _NCHUNK
    gathers[0] = pltpu.async_copy(x_hbm.at[i0], rows0, sg0)
    for t in range(_NCHUNK):
        gathers[t].wait()
        if t >= 1:
            writes[t - 1].wait()
        writes[t] = pltpu.async_copy(rows[t % 2], out_hbm.at[pl.ds(off(t), SUB)],
                                     sw[t % 2])
        if t + 1 < _NCHUNK:
            gathers[t + 1] = pltpu.async_copy(x_hbm.at[idx_bufs[t + 1]],
                                              rows[(t + 1) % 2], sg[(t + 1) % 2])
    writes[_NCHUNK - 1].wait()


@functools.cache
def _sc_gather_kernel():
    return pl.kernel(
        _sc_gather_body,
        out_type=jax.ShapeDtypeStruct((K * N_FINE, C_IN), jnp.float32),
        mesh=plsc.VectorSubcoreMesh(core_axis_name="c", subcore_axis_name="s",
                                    num_cores=NC, num_subcores=NS),
        scratch_types=(
            [pltpu.VMEM((SUB,), jnp.int32) for _ in range(_NCHUNK)]
            + [pltpu.VMEM((SUB, C_IN), jnp.float32) for _ in range(2)]
            + [pltpu.SemaphoreType.DMA for _ in range(5)]
        ),
    )


def _sc_gather(x, idx_flat):
    return _sc_gather_kernel()(x, idx_flat)


def _bn_silu(h, s, ss, g, be):
    mu = s * (1.0 / N_FINE)
    var = ss * (1.0 / N_FINE) - mu * mu
    hn = (h - mu) * lax.rsqrt(var + EPS) * g + be
    return hn * jax.nn.sigmoid(hn)


def _mlp_body(f0_ref, f1_ref, f2_ref, w_ref, xs_ref, W1_ref, b1_ref,
              g1_ref, be1_ref, W2_ref, b2_ref, g2_ref, be2_ref,
              out_ref, h1_s, h2_s, s1_s, ss1_s, s2_s, ss2_s):
    p = pl.program_id(0)
    i = pl.program_id(1)
    row = i * QB

    @pl.when(p == 0)
    def _phase0():
        w = w_ref[...]
        y = (w[:, 0:1] * f0_ref[...] + w[:, 1:2] * f1_ref[...]
             + w[:, 2:3] * f2_ref[...])
        h = (jnp.dot(y, W1_ref[0:C_IN, :],
                     preferred_element_type=jnp.float32)
             + jnp.dot(xs_ref[...], W1_ref[C_IN:C_IN + C_SKIP, :],
                       preferred_element_type=jnp.float32)
             + b1_ref[...])

        @pl.when(i == 0)
        def _init():
            s1_s[...] = jnp.zeros_like(s1_s)
            ss1_s[...] = jnp.zeros_like(ss1_s)

        h1_s[pl.ds(row, QB), :] = h
        s1_s[...] += jnp.sum(h, axis=0, keepdims=True)
        ss1_s[...] += jnp.sum(h * h, axis=0, keepdims=True)

    @pl.when(p == 1)
    def _phase1():
        a = _bn_silu(h1_s[pl.ds(row, QB), :], s1_s[...], ss1_s[...],
                     g1_ref[...], be1_ref[...])
        h = (jnp.dot(a, W2_ref[...], preferred_element_type=jnp.float32)
             + b2_ref[...])

        @pl.when(i == 0)
        def _init():
            s2_s[...] = jnp.zeros_like(s2_s)
            ss2_s[...] = jnp.zeros_like(ss2_s)

        h2_s[pl.ds(row, QB), :] = h
        s2_s[...] += jnp.sum(h, axis=0, keepdims=True)
        ss2_s[...] += jnp.sum(h * h, axis=0, keepdims=True)

    @pl.when(p == 2)
    def _phase2():
        out_ref[...] = _bn_silu(h2_s[pl.ds(row, QB), :], s2_s[...],
                                ss2_s[...], g2_ref[...], be2_ref[...])


def _mlp(feats, w, x_skip, W1, b1, g1, be1, W2, b2, g2, be2):
    def blk(p, i):
        return (jnp.where(p == 0, i, 0), 0)

    return pl.pallas_call(
        _mlp_body,
        grid=(3, NBLK),
        in_specs=[
            pl.BlockSpec((QB, C_IN), lambda p, i: (jnp.where(p == 0, i, 0), 0)),
            pl.BlockSpec((QB, C_IN),
                         lambda p, i: (NBLK + jnp.where(p == 0, i, 0), 0)),
            pl.BlockSpec((QB, C_IN),
                         lambda p, i: (2 * NBLK + jnp.where(p == 0, i, 0), 0)),
            pl.BlockSpec((QB, K), blk),
            pl.BlockSpec((QB, C_SKIP), blk),
            pl.BlockSpec((C_IN + C_SKIP, 256), lambda p, i: (0, 0)),
            pl.BlockSpec((1, 256), lambda p, i: (0, 0)),
            pl.BlockSpec((1, 256), lambda p, i: (0, 0)),
            pl.BlockSpec((1, 256), lambda p, i: (0, 0)),
            pl.BlockSpec((256, 256), lambda p, i: (0, 0)),
            pl.BlockSpec((1, 256), lambda p, i: (0, 0)),
            pl.BlockSpec((1, 256), lambda p, i: (0, 0)),
            pl.BlockSpec((1, 256), lambda p, i: (0, 0)),
        ],
        out_specs=pl.BlockSpec((QB, 256),
                               lambda p, i: (jnp.where(p == 2, i, 0), 0)),
        out_shape=jax.ShapeDtypeStruct((N_FINE, 256), jnp.float32),
        scratch_shapes=[
            pltpu.VMEM((N_FINE, 256), jnp.float32),
            pltpu.VMEM((N_FINE, 256), jnp.float32),
            pltpu.VMEM((1, 256), jnp.float32),
            pltpu.VMEM((1, 256), jnp.float32),
            pltpu.VMEM((1, 256), jnp.float32),
            pltpu.VMEM((1, 256), jnp.float32),
        ],
    )(feats, feats, feats, w, x_skip, W1, b1, g1, be1, W2, b2, g2, be2)


def kernel(x, pos, batch, x_skip, pos_skip, batch_skip,
           W1, b1, g1, be1, W2, b2, g2, be2):
    bs_2d = batch_skip.astype(jnp.int32).reshape(N_FINE, 1)
    bc_2d = batch.astype(jnp.int32).reshape(1, N_COARSE)
    pos_t = pos.T

    idx, w = _knn_topk(pos_skip, bs_2d, pos_t, bc_2d)
    idx_flat = idx.T.reshape(K * N_FINE)

    feats = _sc_gather(x, idx_flat)

    r = lambda v: v.reshape(1, 256)
    h = _mlp(feats, w, x_skip, W1, r(b1), r(g1), r(be1),
             W2, r(b2), r(g2), r(be2))
    return (h, pos_skip, batch_skip)
